# baseline (speedup 1.0000x reference)
import functools

import jax
import jax.numpy as jnp
from jax import lax
from jax.experimental import pallas as pl
from jax.experimental.pallas import tpu as pltpu
from jax.experimental.pallas import tpu_sc as plsc

H = 128
L = 16
NS = 16
BATCH = 4
SEQ = 2048
HSEQ = SEQ // 2
PSLICE = HSEQ // NS

_mesh = plsc.VectorSubcoreMesh(core_axis_name="c", subcore_axis_name="s", num_cores=1)


def _make(tag):
    @functools.partial(
        pl.kernel,
        out_type=jax.ShapeDtypeStruct((BATCH, HSEQ, H), jnp.float32),
        mesh=_mesh,
        scratch_types=[
            [pltpu.VMEM((PSLICE,), jnp.int32) for _ in range(BATCH)],
            pltpu.VMEM((PSLICE, H), jnp.float32),
            [pltpu.VMEM((PSLICE, H), jnp.float32) for _ in range(BATCH)],
            [pltpu.SemaphoreType.DMA for _ in range(BATCH)],
            pltpu.SemaphoreType.DMA,
            pltpu.SemaphoreType.DMA,
            pltpu.SemaphoreType.DMA,
        ],
        name=tag,
    )
    def _half(x_hbm, tok_hbm, pos_hbm, out_hbm,
              idx_bufs, pos_v, tok_bufs, g_sems, idx_sem, pos_sem, out_sem):
        wid = lax.axis_index("s")
        s1 = wid * PSLICE

        idx_copies = [
            pltpu.async_copy(x_hbm.at[b, pl.ds(s1, PSLICE)], idx_bufs[b], idx_sem)
            for b in range(BATCH)
        ]
        pos_copy = pltpu.async_copy(pos_hbm.at[pl.ds(s1, PSLICE), :], pos_v, pos_sem)
        g = []
        for b in range(BATCH):
            idx_copies[b].wait()
            g.append(pltpu.async_copy(tok_hbm.at[idx_bufs[b]], tok_bufs[b], g_sems[b]))
        pos_copy.wait()

        def add_rows(tok_ref):
            @plsc.parallel_loop(0, PSLICE, unroll=2)
            def body(j):
                for c in range(H // L):
                    sl = pl.ds(c * L, L)
                    plsc.addupdate(tok_ref.at[j, sl], pos_v[j, sl])

        outs = []
        for b in range(BATCH):
            g[b].wait()
            add_rows(tok_bufs[b])
            outs.append(pltpu.async_copy(
                tok_bufs[b], out_hbm.at[b, pl.ds(s1, PSLICE), :], out_sem))
        for o in outs:
            o.wait()

    return _half


_half_a = _make("embed_half_a")
_half_b = _make("embed_half_b")


def kernel(x, token_table, position_table):
    xi = x.astype(jnp.int32)
    out_a = _half_a(xi[:, :HSEQ], token_table, position_table[:HSEQ])
    out_b = _half_b(xi[:, HSEQ:], token_table, position_table[HSEQ:])
    return jnp.concatenate([out_a, out_b], axis=1)


# speedup vs baseline: 1.5257x; 1.5257x over previous
import functools

import jax
import jax.numpy as jnp
from jax import lax
from jax.experimental import pallas as pl
from jax.experimental.pallas import tpu as pltpu
from jax.experimental.pallas import tpu_sc as plsc

H = 128
L = 16
NC = 2
NS = 16
NW = NC * NS
BATCH = 4
SEQ = 2048
PSLICE = SEQ // NW

_mesh = plsc.VectorSubcoreMesh(core_axis_name="c", subcore_axis_name="s")


@functools.partial(
    pl.kernel,
    out_type=jax.ShapeDtypeStruct((BATCH, SEQ, H), jnp.float32),
    mesh=_mesh,
    scratch_types=[
        [pltpu.VMEM((PSLICE,), jnp.int32) for _ in range(BATCH)],
        pltpu.VMEM((PSLICE, H), jnp.float32),
        [pltpu.VMEM((PSLICE, H), jnp.float32) for _ in range(BATCH)],
        [pltpu.SemaphoreType.DMA for _ in range(BATCH)],
        pltpu.SemaphoreType.DMA,
        pltpu.SemaphoreType.DMA,
        pltpu.SemaphoreType.DMA,
    ],
)
def _embed_lookup(x_hbm, tok_hbm, pos_hbm, out_hbm,
                  idx_bufs, pos_v, tok_bufs, g_sems, idx_sem, pos_sem, out_sem):
    wid = lax.axis_index("s") * NC + lax.axis_index("c")
    s1 = wid * PSLICE

    idx_copies = [
        pltpu.async_copy(x_hbm.at[b, pl.ds(s1, PSLICE)], idx_bufs[b], idx_sem)
        for b in range(BATCH)
    ]
    pos_copy = pltpu.async_copy(pos_hbm.at[pl.ds(s1, PSLICE), :], pos_v, pos_sem)
    g = []
    for b in range(BATCH):
        idx_copies[b].wait()
        g.append(pltpu.async_copy(tok_hbm.at[idx_bufs[b]], tok_bufs[b], g_sems[b]))
    pos_copy.wait()

    def add_rows(tok_ref):
        @plsc.parallel_loop(0, PSLICE, unroll=2)
        def body(j):
            for c in range(H // L):
                sl = pl.ds(c * L, L)
                plsc.addupdate(tok_ref.at[j, sl], pos_v[j, sl])

    outs = []
    for b in range(BATCH):
        g[b].wait()
        add_rows(tok_bufs[b])
        dst = out_hbm.at[b, pl.ds(s1, PSLICE), :]
        outs.append(pltpu.async_copy(tok_bufs[b], dst, out_sem))
    for o in outs:
        o.wait()


def kernel(x, token_table, position_table):
    return _embed_lookup(x.astype(jnp.int32), token_table, position_table)
